# Initial kernel scaffold; baseline (speedup 1.0000x reference)
import jax
import jax.numpy as jnp
from jax.experimental import pallas as pl


def kernel(x, edge_index, W1, b1, W2, b2):
    raise NotImplementedError("write your pallas kernel here")


# speedup vs baseline: 21.8065x; 21.8065x over previous
import functools

import jax
import jax.numpy as jnp
from jax import lax
from jax.experimental import pallas as pl
from jax.experimental.pallas import tpu as pltpu
from jax.experimental.pallas import tpu_sc as plsc

_NC = 2
_NS = 16
_NW = _NC * _NS
_CH = 128


def _sc_mesh():
    return plsc.VectorSubcoreMesh(core_axis_name="c", subcore_axis_name="s")


@functools.lru_cache(maxsize=None)
def _make_deg_kernel(n_pad, nch):
    rpt = n_pad // _NS

    @functools.partial(
        pl.kernel,
        out_type=jax.ShapeDtypeStruct((_NC * n_pad,), jnp.float32),
        mesh=_sc_mesh(),
        scratch_types=[
            pltpu.VMEM((nch, _CH), jnp.int32),
            pltpu.VMEM((_CH,), jnp.float32),
            pltpu.VMEM((rpt,), jnp.float32),
            pltpu.VMEM_SHARED((n_pad,), jnp.float32),
        ],
    )
    def deg_kernel(dst_hbm, out_hbm, dst_v, ones_v, stage, acc):
        c = lax.axis_index("c")
        s = lax.axis_index("s")
        wid = c * _NS + s
        pltpu.sync_copy(dst_hbm.at[wid], dst_v)
        for i in range(_CH // 16):
            ones_v[pl.ds(i * 16, 16)] = jnp.full((16,), 1.0, jnp.float32)
        for i in range(rpt // 16):
            stage[pl.ds(i * 16, 16)] = jnp.full((16,), 0.0, jnp.float32)
        pltpu.sync_copy(stage, acc.at[pl.ds(s * rpt, rpt)])
        plsc.subcore_barrier()

        def body(j, carry):
            pltpu.sync_copy(ones_v, acc.at[dst_v.at[j]], add=True)
            return carry

        lax.fori_loop(0, nch, body, 0)
        plsc.subcore_barrier()
        pltpu.sync_copy(acc.at[pl.ds(s * rpt, rpt)], stage)
        pltpu.sync_copy(stage, out_hbm.at[pl.ds(c * n_pad + s * rpt, rpt)])

    return deg_kernel


@functools.lru_cache(maxsize=None)
def _make_agg_kernel(n_pad, d, nch):
    rpt = n_pad // _NS
    nrch = rpt // _CH

    @functools.partial(
        pl.kernel,
        out_type=jax.ShapeDtypeStruct((_NC * n_pad, d), jnp.float32),
        mesh=_sc_mesh(),
        scratch_types=[
            pltpu.VMEM((nch, _CH), jnp.int32),
            pltpu.VMEM((nch, _CH), jnp.int32),
            pltpu.VMEM((_CH, d), jnp.float32),
            pltpu.VMEM_SHARED((n_pad, d), jnp.float32),
        ],
    )
    def agg_kernel(g_hbm, src_hbm, dst_hbm, out_hbm, src_v, dst_v, buf, acc):
        c = lax.axis_index("c")
        s = lax.axis_index("s")
        wid = c * _NS + s
        pltpu.sync_copy(src_hbm.at[wid], src_v)
        pltpu.sync_copy(dst_hbm.at[wid], dst_v)
        for i in range(nrch):
            r0 = s * rpt + i * _CH
            pltpu.sync_copy(g_hbm.at[pl.ds(r0, _CH)], buf)
            pltpu.sync_copy(buf, acc.at[pl.ds(r0, _CH)])
        plsc.subcore_barrier()

        def body(j, carry):
            pltpu.sync_copy(g_hbm.at[src_v.at[j]], buf)
            pltpu.sync_copy(buf, acc.at[dst_v.at[j]], add=True)
            return carry

        lax.fori_loop(0, nch, body, 0)
        plsc.subcore_barrier()
        for i in range(nrch):
            r0 = s * rpt + i * _CH
            pltpu.sync_copy(acc.at[pl.ds(r0, _CH)], buf)
            pltpu.sync_copy(buf, out_hbm.at[pl.ds(c * n_pad + r0, _CH)])

    return agg_kernel


def _tc1_body(deg0, deg1, x, w, g_ref, dinv_ref):
    dinv = lax.rsqrt(deg0[...] + deg1[...] + 1.0)
    h = jnp.dot(x[...], w[...], precision=lax.Precision.HIGHEST,
                preferred_element_type=jnp.float32)
    g_ref[...] = h * dinv
    dinv_ref[...] = dinv


def _tc2_body(a0, a1, g1, dinv, b, w, out_ref):
    agg = a0[...] + a1[...] - g1[...]
    h = jnp.maximum(dinv[...] * agg + b[...], 0.0)
    out_ref[...] = dinv[...] * jnp.dot(
        h, w[...], precision=lax.Precision.HIGHEST,
        preferred_element_type=jnp.float32)


def _tc3_body(a0, a1, g2, dinv, b, out_ref):
    logits = dinv[...] * (a0[...] + a1[...] - g2[...]) + b[...]
    m = jnp.max(logits, axis=-1, keepdims=True)
    e = jnp.exp(logits - m)
    out_ref[...] = e / jnp.sum(e, axis=-1, keepdims=True)


def _col_spec(r):
    return pl.BlockSpec((r, 1), lambda i: (i, 0))


def _row_spec(r, d):
    return pl.BlockSpec((r, d), lambda i: (i, 0))


def _row_spec_off(r, d, off_blocks):
    return pl.BlockSpec((r, d), lambda i: (i + off_blocks, 0))


def _full_spec(d0, d1):
    return pl.BlockSpec((d0, d1), lambda i: (0, 0))


def _make_tc1(n_pad, d_in, d_out, r):
    return pl.pallas_call(
        _tc1_body,
        grid=(n_pad // r,),
        in_specs=[_col_spec(r), _col_spec(r), _row_spec(r, d_in),
                  _full_spec(d_in, d_out)],
        out_specs=[_row_spec(r, d_out), _col_spec(r)],
        out_shape=[jax.ShapeDtypeStruct((n_pad, d_out), jnp.float32),
                   jax.ShapeDtypeStruct((n_pad, 1), jnp.float32)],
    )


def _make_tc2(n_pad, d_in, d_out, r):
    nb = n_pad // r
    return pl.pallas_call(
        _tc2_body,
        grid=(nb,),
        in_specs=[_row_spec(r, d_in), _row_spec_off(r, d_in, nb),
                  _row_spec(r, d_in), _col_spec(r), _full_spec(1, d_in),
                  _full_spec(d_in, d_out)],
        out_specs=[_row_spec(r, d_out)],
        out_shape=[jax.ShapeDtypeStruct((n_pad, d_out), jnp.float32)],
    )


def _make_tc3(n_pad, d, r):
    nb = n_pad // r
    return pl.pallas_call(
        _tc3_body,
        grid=(nb,),
        in_specs=[_row_spec(r, d), _row_spec_off(r, d, nb),
                  _row_spec(r, d), _col_spec(r), _full_spec(1, d)],
        out_specs=[_row_spec(r, d)],
        out_shape=[jax.ShapeDtypeStruct((n_pad, d), jnp.float32)],
    )


def kernel(x, edge_index, W1, b1, W2, b2):
    n, d_in = x.shape
    d_hid = W1.shape[1]
    d_out = W2.shape[1]
    e = edge_index.shape[1]

    n_pad = ((n + 16 + 2047) // 2048) * 2048

    nch = -(-e // (_NW * _CH))
    if nch % 2:
        nch += 1
    pad_n = _NW * nch * _CH - e
    pad_i = jnp.arange(pad_n, dtype=jnp.int32)
    src_p = jnp.concatenate([edge_index[0], pad_i % jnp.int32(n)])
    dst_p = jnp.concatenate([edge_index[1], jnp.int32(n) + (pad_i % 16)])
    src_r = src_p.reshape(_NW, nch, _CH)
    dst_r = dst_p.reshape(_NW, nch, _CH)

    x_p = jnp.pad(x, ((0, n_pad - n), (0, 0)))
    b1r = b1.reshape(1, d_hid)
    b2r = b2.reshape(1, d_out)

    deg = _make_deg_kernel(n_pad, nch)(dst_r)
    deg0 = deg[:n_pad].reshape(n_pad, 1)
    deg1 = deg[n_pad:].reshape(n_pad, 1)

    r = n_pad // 8
    g1, dinv = _make_tc1(n_pad, d_in, d_hid, r)(deg0, deg1, x_p, W1)
    agg1 = _make_agg_kernel(n_pad, d_hid, nch)(g1, src_r, dst_r)
    (g2,) = _make_tc2(n_pad, d_hid, d_out, r)(agg1, agg1, g1, dinv, b1r, W2)
    agg2 = _make_agg_kernel(n_pad, d_out, nch)(g2, src_r, dst_r)
    (out,) = _make_tc3(n_pad, d_out, r)(agg2, agg2, g2, dinv, b2r)
    return out[:n]
